# baseline (speedup 1.0000x reference)
import functools

import jax
import jax.numpy as jnp
from jax import lax
from jax.experimental import pallas as pl
from jax.experimental.pallas import tpu as pltpu
from jax.experimental.pallas import tpu_sc as plsc

B, F, L, V, D = 1024, 4, 50, 100000, 128
NC, NS, LANES = 2, 16, 16
NW = NC * NS
PAIRS = F * B
PPW = PAIRS // NW
CROWS = 128
NCHUNK = PPW * L // CROWS
NSUB = D // LANES


def _sc_body(x_hbm, e0, e1, e2, e3, out_hbm, idx_v, buf_v, out_v, sem0, sem1):
  cid = lax.axis_index("c")
  sid = lax.axis_index("s")
  wid = sid * NC + cid
  f = wid // (NW // F)

  pltpu.sync_copy(x_hbm.at[wid], idx_v)

  one = jnp.float32(1.0)
  zero = jnp.float32(0.0)

  def compute(buf, carry):
    def r_body(r, c):
      accs, rcnt, orow = c
      new = []
      for k in range(NSUB):
        v = buf[r, pl.ds(k * LANES, LANES)]
        new.append(accs[k] + v)
      for k in range(NSUB):
        v = buf[r, pl.ds(k * LANES, LANES)]
        new.append(accs[NSUB + k] + jnp.where(v != 0.0, one, zero))

      rcnt = rcnt + 1
      is_end = rcnt == L

      @pl.when(is_end)
      def _():
        for k in range(NSUB):
          out_v[orow, pl.ds(k * LANES, LANES)] = (
              new[k] / (new[NSUB + k] + jnp.float32(1e-16)))

      keep = jnp.full((LANES,), jnp.where(is_end, zero, one))
      accs = tuple(n * keep for n in new)
      rcnt = jnp.where(is_end, 0, rcnt)
      orow = orow + jnp.where(is_end, 1, 0)
      return (accs, rcnt, orow)

    return lax.fori_loop(0, CROWS, r_body, carry)

  def process(table):
    pltpu.async_copy(table.at[idx_v.at[0]], buf_v.at[0], sem0)

    def outer(i, carry):
      j0 = 2 * i
      pltpu.async_copy(table.at[idx_v.at[j0 + 1]], buf_v.at[1], sem1)
      pltpu.make_async_copy(table.at[idx_v.at[j0]], buf_v.at[0], sem0).wait()
      carry = compute(buf_v.at[0], carry)

      @pl.when(i + 1 < NCHUNK // 2)
      def _():
        pltpu.async_copy(table.at[idx_v.at[j0 + 2]], buf_v.at[0], sem0)

      pltpu.make_async_copy(
          table.at[idx_v.at[j0 + 1]], buf_v.at[1], sem1).wait()
      carry = compute(buf_v.at[1], carry)
      return carry

    zeros = tuple(jnp.zeros((LANES,), jnp.float32) for _ in range(2 * NSUB))
    lax.fori_loop(0, NCHUNK // 2, outer,
                  (zeros, jnp.int32(0), jnp.int32(0)))

  @pl.when(f == 0)
  def _():
    process(e0)

  @pl.when(f == 1)
  def _():
    process(e1)

  @pl.when(f == 2)
  def _():
    process(e2)

  @pl.when(f == 3)
  def _():
    process(e3)

  b0 = (wid % (NW // F)) * PPW
  pltpu.sync_copy(out_v, out_hbm.at[pl.ds(b0, PPW), f])


@jax.jit
def kernel(x, emb0, emb1, emb2, emb3):
  xt = jnp.transpose(x, (1, 0, 2)).reshape(NW, NCHUNK, CROWS)

  mesh = plsc.VectorSubcoreMesh(core_axis_name="c", subcore_axis_name="s")
  out = pl.kernel(
      _sc_body,
      out_type=jax.ShapeDtypeStruct((B, F, D), jnp.float32),
      mesh=mesh,
      scratch_types=[
          pltpu.VMEM((NCHUNK, CROWS), jnp.int32),
          pltpu.VMEM((2, CROWS, D), jnp.float32),
          pltpu.VMEM((PPW, D), jnp.float32),
          pltpu.SemaphoreType.DMA,
          pltpu.SemaphoreType.DMA,
      ],
  )(xt, emb0, emb1, emb2, emb3)

  return out


# speedup vs baseline: 1.8241x; 1.8241x over previous
import functools

import jax
import jax.numpy as jnp
from jax import lax
from jax.experimental import pallas as pl
from jax.experimental.pallas import tpu as pltpu
from jax.experimental.pallas import tpu_sc as plsc

B, F, L, V, D = 1024, 4, 50, 100000, 128
NC, NS, LANES = 2, 16, 16
NW = NC * NS
PAIRS = F * B
PPW = PAIRS // NW
CROWS = 128
NCHUNK = PPW * L // CROWS
NSUB = D // LANES


def _sc_body(x_hbm, e0, e1, e2, e3, out_hbm, idx_v, buf_v, out_v, sem0, sem1):
  cid = lax.axis_index("c")
  sid = lax.axis_index("s")
  wid = sid * NC + cid
  f = wid // (NW // F)

  pltpu.sync_copy(x_hbm.at[wid], idx_v)

  one = jnp.float32(1.0)
  zero = jnp.float32(0.0)
  zeros = tuple(jnp.zeros((LANES,), jnp.float32) for _ in range(2 * NSUB))

  def compute(buf, carry):
    accs, rem, orow = carry

    def run(accs, s, e):
      def r_body(r, a):
        vs = [buf[r, pl.ds(k * LANES, LANES)] for k in range(NSUB)]
        news = [a[k] + vs[k] for k in range(NSUB)]
        newc = [a[NSUB + k] + jnp.where(vs[k] != 0.0, one, zero)
                for k in range(NSUB)]
        return tuple(news + newc)

      return lax.fori_loop(s, e, r_body, accs)

    def store(row, a):
      for k in range(NSUB):
        out_v[row, pl.ds(k * LANES, LANES)] = (
            a[k] / (a[NSUB + k] + jnp.float32(1e-16)))

    e1 = rem
    accs = run(accs, 0, e1)
    store(orow, accs)
    accs = run(zeros, e1, e1 + L)
    store(orow + 1, accs)
    e3 = jnp.minimum(e1 + 2 * L, CROWS)
    accs = run(zeros, e1 + L, e3)
    full3 = e1 + 2 * L <= CROWS

    @pl.when(full3)
    def _():
      store(orow + 2, accs)

    keep = jnp.full((LANES,), jnp.where(full3, zero, one))
    accs = tuple(a * keep for a in accs)
    accs = run(accs, e3, CROWS)

    orow = orow + jnp.where(full3, 3, 2)
    rem = jnp.where(full3, e1 + (3 * L - CROWS), e1 + (2 * L - CROWS))
    return (accs, rem, orow)

  def process(table):
    pltpu.async_copy(table.at[idx_v.at[0]], buf_v.at[0], sem0)

    def outer(i, carry):
      j0 = 2 * i
      pltpu.async_copy(table.at[idx_v.at[j0 + 1]], buf_v.at[1], sem1)
      pltpu.make_async_copy(table.at[idx_v.at[j0]], buf_v.at[0], sem0).wait()
      carry = compute(buf_v.at[0], carry)

      @pl.when(i + 1 < NCHUNK // 2)
      def _():
        pltpu.async_copy(table.at[idx_v.at[j0 + 2]], buf_v.at[0], sem0)

      pltpu.make_async_copy(
          table.at[idx_v.at[j0 + 1]], buf_v.at[1], sem1).wait()
      carry = compute(buf_v.at[1], carry)
      return carry

    lax.fori_loop(0, NCHUNK // 2, outer,
                  (zeros, jnp.int32(L), jnp.int32(0)))

  @pl.when(f == 0)
  def _():
    process(e0)

  @pl.when(f == 1)
  def _():
    process(e1)

  @pl.when(f == 2)
  def _():
    process(e2)

  @pl.when(f == 3)
  def _():
    process(e3)

  b0 = (wid % (NW // F)) * PPW
  pltpu.sync_copy(out_v, out_hbm.at[pl.ds(b0, PPW), f])


@jax.jit
def kernel(x, emb0, emb1, emb2, emb3):
  xt = jnp.transpose(x, (1, 0, 2)).reshape(NW, NCHUNK, CROWS)

  mesh = plsc.VectorSubcoreMesh(core_axis_name="c", subcore_axis_name="s")
  out = pl.kernel(
      _sc_body,
      out_type=jax.ShapeDtypeStruct((B, F, D), jnp.float32),
      mesh=mesh,
      scratch_types=[
          pltpu.VMEM((NCHUNK, CROWS), jnp.int32),
          pltpu.VMEM((2, CROWS, D), jnp.float32),
          pltpu.VMEM((PPW, D), jnp.float32),
          pltpu.SemaphoreType.DMA,
          pltpu.SemaphoreType.DMA,
      ],
  )(xt, emb0, emb1, emb2, emb3)

  return out


# speedup vs baseline: 1.8303x; 1.0034x over previous
import functools

import jax
import jax.numpy as jnp
from jax import lax
from jax.experimental import pallas as pl
from jax.experimental.pallas import tpu as pltpu
from jax.experimental.pallas import tpu_sc as plsc

B, F, L, V, D = 1024, 4, 50, 100000, 128
NC, NS, LANES = 2, 16, 16
NW = NC * NS
PAIRS = F * B
PPW = PAIRS // NW
CROWS = 128
NCHUNK = PPW * L // CROWS
NSUB = D // LANES


def _sc_body(x_hbm, e0, e1, e2, e3, out_hbm, idx_v, buf_v, out_v, sem0, sem1):
  cid = lax.axis_index("c")
  sid = lax.axis_index("s")
  wid = sid * NC + cid
  f = wid // (NW // F)

  pltpu.sync_copy(x_hbm.at[wid], idx_v)

  one = jnp.float32(1.0)
  zero = jnp.float32(0.0)
  zeros = tuple(jnp.zeros((LANES,), jnp.float32) for _ in range(2 * NSUB))

  def compute(buf, carry):
    accs, rem, orow = carry

    def run(accs, s, e):
      def r_body(r, a):
        vs = [buf[r, pl.ds(k * LANES, LANES)] for k in range(NSUB)]
        news = [a[k] + vs[k] for k in range(NSUB)]
        newc = [a[NSUB + k] + jnp.where(vs[k] != 0.0, one, zero)
                for k in range(NSUB)]
        return tuple(news + newc)

      return plsc.parallel_loop(s, e, carry=accs, unroll=2)(r_body)

    def store(row, a):
      for k in range(NSUB):
        out_v[row, pl.ds(k * LANES, LANES)] = (
            a[k] / (a[NSUB + k] + jnp.float32(1e-16)))

    e1 = rem
    accs = run(accs, 0, e1)
    store(orow, accs)
    accs = run(zeros, e1, e1 + L)
    store(orow + 1, accs)
    e3 = jnp.minimum(e1 + 2 * L, CROWS)
    accs = run(zeros, e1 + L, e3)
    full3 = e1 + 2 * L <= CROWS

    @pl.when(full3)
    def _():
      store(orow + 2, accs)

    keep = jnp.full((LANES,), jnp.where(full3, zero, one))
    accs = tuple(a * keep for a in accs)
    accs = run(accs, e3, CROWS)

    orow = orow + jnp.where(full3, 3, 2)
    rem = jnp.where(full3, e1 + (3 * L - CROWS), e1 + (2 * L - CROWS))
    return (accs, rem, orow)

  def process(table):
    pltpu.async_copy(table.at[idx_v.at[0]], buf_v.at[0], sem0)

    def outer(i, carry):
      j0 = 2 * i
      pltpu.async_copy(table.at[idx_v.at[j0 + 1]], buf_v.at[1], sem1)
      pltpu.make_async_copy(table.at[idx_v.at[j0]], buf_v.at[0], sem0).wait()
      carry = compute(buf_v.at[0], carry)

      @pl.when(i + 1 < NCHUNK // 2)
      def _():
        pltpu.async_copy(table.at[idx_v.at[j0 + 2]], buf_v.at[0], sem0)

      pltpu.make_async_copy(
          table.at[idx_v.at[j0 + 1]], buf_v.at[1], sem1).wait()
      carry = compute(buf_v.at[1], carry)
      return carry

    lax.fori_loop(0, NCHUNK // 2, outer,
                  (zeros, jnp.int32(L), jnp.int32(0)))

  @pl.when(f == 0)
  def _():
    process(e0)

  @pl.when(f == 1)
  def _():
    process(e1)

  @pl.when(f == 2)
  def _():
    process(e2)

  @pl.when(f == 3)
  def _():
    process(e3)

  b0 = (wid % (NW // F)) * PPW
  pltpu.sync_copy(out_v, out_hbm.at[pl.ds(b0, PPW), f])


@jax.jit
def kernel(x, emb0, emb1, emb2, emb3):
  xt = jnp.transpose(x, (1, 0, 2)).reshape(NW, NCHUNK, CROWS)

  mesh = plsc.VectorSubcoreMesh(core_axis_name="c", subcore_axis_name="s")
  out = pl.kernel(
      _sc_body,
      out_type=jax.ShapeDtypeStruct((B, F, D), jnp.float32),
      mesh=mesh,
      scratch_types=[
          pltpu.VMEM((NCHUNK, CROWS), jnp.int32),
          pltpu.VMEM((2, CROWS, D), jnp.float32),
          pltpu.VMEM((PPW, D), jnp.float32),
          pltpu.SemaphoreType.DMA,
          pltpu.SemaphoreType.DMA,
      ],
  )(xt, emb0, emb1, emb2, emb3)

  return out


# speedup vs baseline: 1.8346x; 1.0023x over previous
import functools

import jax
import jax.numpy as jnp
from jax import lax
from jax.experimental import pallas as pl
from jax.experimental.pallas import tpu as pltpu
from jax.experimental.pallas import tpu_sc as plsc

B, F, L, V, D = 1024, 4, 50, 100000, 128
NC, NS, LANES = 2, 16, 16
NW = NC * NS
PAIRS = F * B
PPW = PAIRS // NW
CROWS = 128
NCHUNK = PPW * L // CROWS
NSUB = D // LANES


def _sc_body(x_hbm, e0, e1, e2, e3, out_hbm, idx_v, buf_v, out_v, sem0, sem1):
  cid = lax.axis_index("c")
  sid = lax.axis_index("s")
  wid = sid * NC + cid
  f = wid // (NW // F)

  pltpu.sync_copy(x_hbm.at[wid], idx_v)

  one = jnp.float32(1.0)
  zero = jnp.float32(0.0)
  zeros = tuple(jnp.zeros((LANES,), jnp.float32) for _ in range(2 * NSUB))

  def compute(buf, carry):
    accs, rem, orow = carry

    def run(accs, s, e):
      def r_body(r, a):
        vs = [buf[r, pl.ds(k * LANES, LANES)] for k in range(NSUB)]
        news = [a[k] + vs[k] for k in range(NSUB)]
        newc = [a[NSUB + k] + jnp.where(vs[k] != 0.0, one, zero)
                for k in range(NSUB)]
        return tuple(news + newc)

      return plsc.parallel_loop(s, e, carry=accs, unroll=4)(r_body)

    def store(row, a):
      for k in range(NSUB):
        out_v[row, pl.ds(k * LANES, LANES)] = (
            a[k] / (a[NSUB + k] + jnp.float32(1e-16)))

    e1 = rem
    accs = run(accs, 0, e1)
    store(orow, accs)
    accs = run(zeros, e1, e1 + L)
    store(orow + 1, accs)
    e3 = jnp.minimum(e1 + 2 * L, CROWS)
    accs = run(zeros, e1 + L, e3)
    full3 = e1 + 2 * L <= CROWS

    @pl.when(full3)
    def _():
      store(orow + 2, accs)

    keep = jnp.full((LANES,), jnp.where(full3, zero, one))
    accs = tuple(a * keep for a in accs)
    accs = run(accs, e3, CROWS)

    orow = orow + jnp.where(full3, 3, 2)
    rem = jnp.where(full3, e1 + (3 * L - CROWS), e1 + (2 * L - CROWS))
    return (accs, rem, orow)

  def process(table):
    pltpu.async_copy(table.at[idx_v.at[0]], buf_v.at[0], sem0)

    def outer(i, carry):
      j0 = 2 * i
      pltpu.async_copy(table.at[idx_v.at[j0 + 1]], buf_v.at[1], sem1)
      pltpu.make_async_copy(table.at[idx_v.at[j0]], buf_v.at[0], sem0).wait()
      carry = compute(buf_v.at[0], carry)

      @pl.when(i + 1 < NCHUNK // 2)
      def _():
        pltpu.async_copy(table.at[idx_v.at[j0 + 2]], buf_v.at[0], sem0)

      pltpu.make_async_copy(
          table.at[idx_v.at[j0 + 1]], buf_v.at[1], sem1).wait()
      carry = compute(buf_v.at[1], carry)
      return carry

    lax.fori_loop(0, NCHUNK // 2, outer,
                  (zeros, jnp.int32(L), jnp.int32(0)))

  @pl.when(f == 0)
  def _():
    process(e0)

  @pl.when(f == 1)
  def _():
    process(e1)

  @pl.when(f == 2)
  def _():
    process(e2)

  @pl.when(f == 3)
  def _():
    process(e3)

  b0 = (wid % (NW // F)) * PPW
  pltpu.sync_copy(out_v, out_hbm.at[pl.ds(b0, PPW), f])


@jax.jit
def kernel(x, emb0, emb1, emb2, emb3):
  xt = jnp.transpose(x, (1, 0, 2)).reshape(NW, NCHUNK, CROWS)

  mesh = plsc.VectorSubcoreMesh(core_axis_name="c", subcore_axis_name="s")
  out = pl.kernel(
      _sc_body,
      out_type=jax.ShapeDtypeStruct((B, F, D), jnp.float32),
      mesh=mesh,
      scratch_types=[
          pltpu.VMEM((NCHUNK, CROWS), jnp.int32),
          pltpu.VMEM((2, CROWS, D), jnp.float32),
          pltpu.VMEM((PPW, D), jnp.float32),
          pltpu.SemaphoreType.DMA,
          pltpu.SemaphoreType.DMA,
      ],
  )(xt, emb0, emb1, emb2, emb3)

  return out
